# baseline (speedup 1.0000x reference)
import functools

import jax
import jax.numpy as jnp
from jax import lax
from jax.experimental import pallas as pl
from jax.experimental.pallas import tpu as pltpu
from jax.experimental.pallas import tpu_sc as plsc

N_ROWS = 100000
D_FEAT = 128
N_SEG = 512
LANES = 16
NV = D_FEAT // LANES
N_WORKERS = 32
SEGS_PER_W = N_SEG // N_WORKERS
TILE = 128
NBUF = 3


@functools.partial(
    pl.kernel,
    mesh=plsc.VectorSubcoreMesh(core_axis_name="c", subcore_axis_name="s"),
    out_type=jax.ShapeDtypeStruct((N_SEG, 3 * D_FEAT), jnp.float32),
    scratch_types=[
        pltpu.VMEM((2 * LANES,), jnp.int32),
        pltpu.SMEM((2 * LANES,), jnp.int32),
        pltpu.VMEM((NBUF, TILE, D_FEAT), jnp.float32),
        pltpu.VMEM((2, D_FEAT), jnp.float32),
        pltpu.VMEM((SEGS_PER_W, 3 * D_FEAT), jnp.float32),
        pltpu.SemaphoreType.DMA,
        pltpu.SemaphoreType.DMA,
        pltpu.SemaphoreType.DMA,
    ],
)
def _pool_sc(x_hbm, off_hbm, out_hbm, offs_v, offs_s, ring, accv, outbuf,
             sem0, sem1, sem2):
    sems = (sem0, sem1, sem2)
    wid = lax.axis_index("s") * 2 + lax.axis_index("c")
    seg0 = wid * SEGS_PER_W

    pltpu.sync_copy(off_hbm.at[pl.ds(seg0, 2 * LANES)], offs_v)
    v_s = offs_v[pl.ds(0, LANES)]
    v_e = offs_v[pl.ds(LANES, LANES)]
    for jj in range(LANES):
        offs_s[jj] = v_s[jj]
        offs_s[LANES + jj] = v_e[jj]

    def s_of(t, base):
        return pl.multiple_of(
            jnp.minimum(base + t * TILE, N_ROWS - TILE), 8)

    def issue(t, b, base):
        pltpu.async_copy(
            x_hbm.at[pl.ds(s_of(t, base), TILE)], ring.at[b], sems[b])

    def drain(b):
        pltpu.make_async_copy(
            x_hbm.at[pl.ds(0, TILE)], ring.at[b], sems[b]).wait()

    RSUB = 8

    def accum_fast(buf):
        def sub(k, carry):
            accs = list(carry)
            for rr in range(RSUB):
                for v in range(NV):
                    xv = buf[k * RSUB + rr, pl.ds(v * LANES, LANES)]
                    accs[v] = accs[v] + xv
                    accs[NV + v] = jnp.maximum(accs[NV + v], xv)
            return tuple(accs)

        init = tuple(accv[0, pl.ds(v * LANES, LANES)] for v in range(NV)) +                tuple(accv[1, pl.ds(v * LANES, LANES)] for v in range(NV))
        accs = lax.fori_loop(0, TILE // RSUB, sub, init)
        for v in range(NV):
            accv[0, pl.ds(v * LANES, LANES)] = accs[v]
            accv[1, pl.ds(v * LANES, LANES)] = accs[NV + v]

    def accum_masked(buf, lo, hi):
        def sub(k, carry):
            accs = list(carry)
            for rr in range(RSUB):
                r = k * RSUB + rr
                rv = (r >= lo) & (r < hi)
                for v in range(NV):
                    xv = buf[r, pl.ds(v * LANES, LANES)]
                    accs[v] = accs[v] + jnp.where(rv, xv, 0.0)
                    accs[NV + v] = jnp.maximum(
                        accs[NV + v], jnp.where(rv, xv, -jnp.inf))
            return tuple(accs)

        init = tuple(accv[0, pl.ds(v * LANES, LANES)] for v in range(NV)) +                tuple(accv[1, pl.ds(v * LANES, LANES)] for v in range(NV))
        accs = lax.fori_loop(0, TILE // RSUB, sub, init)
        for v in range(NV):
            accv[0, pl.ds(v * LANES, LANES)] = accs[v]
            accv[1, pl.ds(v * LANES, LANES)] = accs[NV + v]

    zero = jnp.zeros((LANES,), jnp.float32)
    ninf = jnp.full((LANES,), -jnp.inf, jnp.float32)

    def seg_body(j, seg_carry):
        start = offs_s[j]
        end = offs_s[j + 1]
        n = end - start
        base = start & ~7
        num_tiles = (end - base + TILE - 1) // TILE
        num_groups = (num_tiles + NBUF - 1) // NBUF

        for v in range(NV):
            accv[0, pl.ds(v * LANES, LANES)] = zero
            accv[1, pl.ds(v * LANES, LANES)] = ninf

        for b in range(NBUF):
            @pl.when(b < num_tiles)
            def _(b=b, base=base):
                issue(b, b, base)

        def group_body(p, carry, start=start, end=end, base=base,
                       num_tiles=num_tiles):
            for b in range(NBUF):
                t = p * NBUF + b
                s_t = s_of(t, base)
                tbase = base + t * TILE
                lo = jnp.maximum(start, tbase) - s_t
                hi = end - s_t
                full = (lo == 0) & (hi >= TILE)

                @pl.when(t < num_tiles)
                def _(b=b, lo=lo, hi=hi, full=full):
                    drain(b)

                    @pl.when(full)
                    def _():
                        accum_fast(ring.at[b])

                    @pl.when(jnp.logical_not(full))
                    def _():
                        accum_masked(ring.at[b], lo, hi)

                @pl.when(t + NBUF < num_tiles)
                def _(t=t, b=b, base=base):
                    issue(t + NBUF, b, base)
            return carry

        lax.fori_loop(0, num_groups, group_body, 0)

        nf = jnp.maximum(n, 1).astype(jnp.float32)
        nonempty = n > 0
        for v in range(NV):
            sv = accv[0, pl.ds(v * LANES, LANES)]
            mv = accv[1, pl.ds(v * LANES, LANES)]
            outbuf[j, pl.ds(v * LANES, LANES)] = sv / nf
            outbuf[j, pl.ds(D_FEAT + v * LANES, LANES)] = jnp.where(
                nonempty, mv, 0.0)
            outbuf[j, pl.ds(2 * D_FEAT + v * LANES, LANES)] = sv
        return seg_carry

    lax.fori_loop(0, SEGS_PER_W, seg_body, 0)

    pltpu.sync_copy(outbuf, out_hbm.at[pl.ds(seg0, SEGS_PER_W)])


def kernel(x, batch):
    batch32 = batch.astype(jnp.int32)
    offsets = jnp.searchsorted(
        batch32, jnp.arange(N_SEG + 1, dtype=jnp.int32),
        method="scan_unrolled").astype(jnp.int32)
    offsets = jnp.pad(offsets, (0, 2 * LANES - 1))
    return _pool_sc(x, offsets)


# speedup vs baseline: 1.1786x; 1.1786x over previous
import functools

import jax
import jax.numpy as jnp
from jax import lax
from jax.experimental import pallas as pl
from jax.experimental.pallas import tpu as pltpu
from jax.experimental.pallas import tpu_sc as plsc

N_ROWS = 100000
D_FEAT = 128
N_SEG = 512
LANES = 16
NV = D_FEAT // LANES
N_WORKERS = 32
SEGS_PER_W = N_SEG // N_WORKERS
TILE = 64
NBUF = 3


@functools.partial(
    pl.kernel,
    mesh=plsc.VectorSubcoreMesh(core_axis_name="c", subcore_axis_name="s"),
    out_type=jax.ShapeDtypeStruct((N_SEG, 3 * D_FEAT), jnp.float32),
    scratch_types=[
        pltpu.VMEM((2 * LANES,), jnp.int32),
        pltpu.SMEM((2 * LANES,), jnp.int32),
        pltpu.VMEM((NBUF, TILE, D_FEAT), jnp.float32),
        pltpu.VMEM((2, D_FEAT), jnp.float32),
        pltpu.VMEM((SEGS_PER_W, 3 * D_FEAT), jnp.float32),
        pltpu.SemaphoreType.DMA,
        pltpu.SemaphoreType.DMA,
        pltpu.SemaphoreType.DMA,
    ],
)
def _pool_sc(x_hbm, off_hbm, out_hbm, offs_v, offs_s, ring, accv, outbuf,
             sem0, sem1, sem2):
    sems = (sem0, sem1, sem2)
    wid = lax.axis_index("s") * 2 + lax.axis_index("c")
    seg0 = wid * SEGS_PER_W

    pltpu.sync_copy(off_hbm.at[pl.ds(seg0, 2 * LANES)], offs_v)
    v_s = offs_v[pl.ds(0, LANES)]
    v_e = offs_v[pl.ds(LANES, LANES)]
    for jj in range(LANES):
        offs_s[jj] = v_s[jj]
        offs_s[LANES + jj] = v_e[jj]

    def s_of(t, base):
        return pl.multiple_of(
            jnp.minimum(base + t * TILE, N_ROWS - TILE), 8)

    def issue(t, b, base):
        pltpu.async_copy(
            x_hbm.at[pl.ds(s_of(t, base), TILE)], ring.at[b], sems[b])

    def drain(b):
        pltpu.make_async_copy(
            x_hbm.at[pl.ds(0, TILE)], ring.at[b], sems[b]).wait()

    RSUB = 8

    def accum_fast(buf):
        def sub(k, carry):
            accs = list(carry)
            for rr in range(RSUB):
                for v in range(NV):
                    xv = buf[k * RSUB + rr, pl.ds(v * LANES, LANES)]
                    accs[v] = accs[v] + xv
                    accs[NV + v] = jnp.maximum(accs[NV + v], xv)
            return tuple(accs)

        init = tuple(accv[0, pl.ds(v * LANES, LANES)] for v in range(NV)) +                tuple(accv[1, pl.ds(v * LANES, LANES)] for v in range(NV))
        accs = lax.fori_loop(0, TILE // RSUB, sub, init)
        for v in range(NV):
            accv[0, pl.ds(v * LANES, LANES)] = accs[v]
            accv[1, pl.ds(v * LANES, LANES)] = accs[NV + v]

    def accum_masked(buf, lo, hi):
        lo_c = jnp.maximum(lo, 0)
        hi_c = jnp.minimum(hi, TILE)
        def sub(k, carry):
            accs = list(carry)
            for rr in range(RSUB):
                r = k * RSUB + rr
                rv = (r >= lo) & (r < hi)
                for v in range(NV):
                    xv = buf[r, pl.ds(v * LANES, LANES)]
                    accs[v] = accs[v] + jnp.where(rv, xv, 0.0)
                    accs[NV + v] = jnp.maximum(
                        accs[NV + v], jnp.where(rv, xv, -jnp.inf))
            return tuple(accs)

        init = tuple(accv[0, pl.ds(v * LANES, LANES)] for v in range(NV)) +                tuple(accv[1, pl.ds(v * LANES, LANES)] for v in range(NV))
        accs = lax.fori_loop(
            lo_c // RSUB, (hi_c + RSUB - 1) // RSUB, sub, init)
        for v in range(NV):
            accv[0, pl.ds(v * LANES, LANES)] = accs[v]
            accv[1, pl.ds(v * LANES, LANES)] = accs[NV + v]

    zero = jnp.zeros((LANES,), jnp.float32)
    ninf = jnp.full((LANES,), -jnp.inf, jnp.float32)

    def seg_body(j, seg_carry):
        start = offs_s[j]
        end = offs_s[j + 1]
        n = end - start
        base = start & ~7
        num_tiles = (end - base + TILE - 1) // TILE
        num_groups = (num_tiles + NBUF - 1) // NBUF

        for v in range(NV):
            accv[0, pl.ds(v * LANES, LANES)] = zero
            accv[1, pl.ds(v * LANES, LANES)] = ninf

        for b in range(NBUF):
            @pl.when(b < num_tiles)
            def _(b=b, base=base):
                issue(b, b, base)

        def group_body(p, carry, start=start, end=end, base=base,
                       num_tiles=num_tiles):
            for b in range(NBUF):
                t = p * NBUF + b
                s_t = s_of(t, base)
                tbase = base + t * TILE
                lo = jnp.maximum(start, tbase) - s_t
                hi = end - s_t
                full = (lo == 0) & (hi >= TILE)

                @pl.when(t < num_tiles)
                def _(b=b, lo=lo, hi=hi, full=full):
                    drain(b)

                    @pl.when(full)
                    def _():
                        accum_fast(ring.at[b])

                    @pl.when(jnp.logical_not(full))
                    def _():
                        accum_masked(ring.at[b], lo, hi)

                @pl.when(t + NBUF < num_tiles)
                def _(t=t, b=b, base=base):
                    issue(t + NBUF, b, base)
            return carry

        lax.fori_loop(0, num_groups, group_body, 0)

        nf = jnp.maximum(n, 1).astype(jnp.float32)
        nonempty = n > 0
        for v in range(NV):
            sv = accv[0, pl.ds(v * LANES, LANES)]
            mv = accv[1, pl.ds(v * LANES, LANES)]
            outbuf[j, pl.ds(v * LANES, LANES)] = sv / nf
            outbuf[j, pl.ds(D_FEAT + v * LANES, LANES)] = jnp.where(
                nonempty, mv, 0.0)
            outbuf[j, pl.ds(2 * D_FEAT + v * LANES, LANES)] = sv
        return seg_carry

    lax.fori_loop(0, SEGS_PER_W, seg_body, 0)

    pltpu.sync_copy(outbuf, out_hbm.at[pl.ds(seg0, SEGS_PER_W)])


def kernel(x, batch):
    batch32 = batch.astype(jnp.int32)
    offsets = jnp.searchsorted(
        batch32, jnp.arange(N_SEG + 1, dtype=jnp.int32),
        method="scan_unrolled").astype(jnp.int32)
    offsets = jnp.pad(offsets, (0, 2 * LANES - 1))
    return _pool_sc(x, offsets)
